# Initial kernel scaffold; baseline (speedup 1.0000x reference)
import jax
import jax.numpy as jnp
from jax.experimental import pallas as pl


def kernel(idx, tables, W1, b1, g1a, be1a, g1b, be1b, W2, b2, g2a, be2a, g2b, be2b, W3, b3):
    raise NotImplementedError("write your pallas kernel here")


# speedup vs baseline: 2.1685x; 2.1685x over previous
import functools

import jax
import jax.numpy as jnp
from jax import lax
from jax.experimental import pallas as pl
from jax.experimental.pallas import tpu as pltpu
from jax.experimental.pallas import tpu_sc as plsc

G = 26
V = 100000
D = 16
B = 16384
GD = G * D
H1 = 256
H2 = 128
EPS = 1e-5

_NC = 2
_NS = 16
_NW = _NC * _NS
_NIDX = B * G
_PER_W = _NIDX // _NW
_ROWS = _PER_W // 128
_GK = 8
_NGRP = _ROWS // _GK
_GROWS = _GK * 128


def _gather_body(idx_hbm, tab_hbm, out_hbm, idx_v, rows_v, sem_g, sem_o):
    wid = lax.axis_index("s") * _NC + lax.axis_index("c")
    rbase = wid * _ROWS
    nbase = wid * _PER_W
    pltpu.sync_copy(idx_hbm.at[pl.ds(rbase, _ROWS)], idx_v)

    iota = lax.iota(jnp.int32, 16)

    def _off(r, carry):
        for k in range(8):
            pos = (r * 128 + k * 16) + iota
            fld = pos % G
            idx_v[r, pl.ds(k * 16, 16)] = idx_v[r, pl.ds(k * 16, 16)] + fld * V
        return carry

    lax.fori_loop(0, _ROWS, _off, 0)

    def _grp(g, carry):
        s = (g % 2) * _GROWS

        @pl.when(g >= 2)
        def _():
            pltpu.make_async_copy(tab_hbm.at[pl.ds(0, _GROWS)],
                                  rows_v.at[pl.ds(0, _GROWS)], sem_o).wait()

        for q in range(_GK):
            r = g * _GK + q
            pltpu.async_copy(tab_hbm.at[idx_v.at[r]],
                             rows_v.at[pl.ds(s + q * 128, 128)], sem_g)
        pltpu.make_async_copy(tab_hbm.at[pl.ds(0, _GROWS)],
                              rows_v.at[pl.ds(0, _GROWS)], sem_g).wait()
        pltpu.async_copy(rows_v.at[pl.ds(s, _GROWS)],
                         out_hbm.at[pl.ds(nbase + g * _GROWS, _GROWS)], sem_o)
        return carry

    lax.fori_loop(0, _NGRP, _grp, 0)
    for _ in range(2):
        pltpu.make_async_copy(tab_hbm.at[pl.ds(0, _GROWS)],
                              rows_v.at[pl.ds(0, _GROWS)], sem_o).wait()


@functools.cache
def _mk_gather():
    return functools.partial(
        pl.kernel,
        out_type=jax.ShapeDtypeStruct((_NIDX, D), jnp.float32),
        mesh=plsc.VectorSubcoreMesh(core_axis_name="c", subcore_axis_name="s",
                                    num_cores=_NC, num_subcores=_NS),
        scratch_types=[
            pltpu.VMEM((_ROWS, 128), jnp.int32),
            pltpu.VMEM((2 * _GROWS, D), jnp.float32),
            pltpu.SemaphoreType.DMA,
            pltpu.SemaphoreType.DMA,
        ],
        compiler_params=pltpu.CompilerParams(use_tc_tiling_on_sc=False),
    )(_gather_body)


TB = 2048
NT = B // TB


def _mlp_body(x_ref, W1_ref, b1_ref, g1a_ref, be1a_ref, g1b_ref, be1b_ref,
              W2_ref, b2_ref, g2a_ref, be2a_ref, g2b_ref, be2b_ref,
              W3_ref, b3_ref, out_ref,
              h1_ref, h2_ref, s1_ref, s2_ref, a1_ref, a2_ref):
    p = pl.program_id(0)
    i = pl.program_id(1)

    def _affine(s_ref, ga, ba, gb, bb, a_ref):
        n = jnp.float32(B)
        m = s_ref[0:1, :] / n
        v = s_ref[1:2, :] / n - m * m
        vy = (ga * ga) * v / (v + EPS)
        scale = ga * gb * lax.rsqrt(v + EPS) * lax.rsqrt(vy + EPS)
        a_ref[0:1, :] = scale
        a_ref[1:2, :] = bb - m * scale

    @pl.when(p == 0)
    def _p0():
        @pl.when(i == 0)
        def _():
            s1_ref[...] = jnp.zeros_like(s1_ref)

        h = jnp.dot(x_ref[...], W1_ref[...],
                    preferred_element_type=jnp.float32) + b1_ref[...]
        h1_ref[pl.ds(i * TB, TB), :] = h
        s1_ref[0:1, :] += jnp.sum(h, axis=0, keepdims=True)
        s1_ref[1:2, :] += jnp.sum(h * h, axis=0, keepdims=True)

        @pl.when(i == NT - 1)
        def _():
            _affine(s1_ref, g1a_ref[...], be1a_ref[...],
                    g1b_ref[...], be1b_ref[...], a1_ref)

    @pl.when(p == 1)
    def _p1():
        @pl.when(i == 0)
        def _():
            s2_ref[...] = jnp.zeros_like(s2_ref)

        h = h1_ref[pl.ds(i * TB, TB), :]
        y = jnp.maximum(h * a1_ref[0:1, :] + a1_ref[1:2, :], 0.0)
        h2 = jnp.dot(y, W2_ref[...],
                     preferred_element_type=jnp.float32) + b2_ref[...]
        h2_ref[pl.ds(i * TB, TB), :] = h2
        s2_ref[0:1, :] += jnp.sum(h2, axis=0, keepdims=True)
        s2_ref[1:2, :] += jnp.sum(h2 * h2, axis=0, keepdims=True)

        @pl.when(i == NT - 1)
        def _():
            _affine(s2_ref, g2a_ref[...], be2a_ref[...],
                    g2b_ref[...], be2b_ref[...], a2_ref)

    @pl.when(p == 2)
    def _p2():
        h = h2_ref[pl.ds(i * TB, TB), :]
        y = jnp.maximum(h * a2_ref[0:1, :] + a2_ref[1:2, :], 0.0)
        z = jnp.dot(y, W3_ref[...],
                    preferred_element_type=jnp.float32) + b3_ref[...]
        out_ref[...] = jax.nn.sigmoid(z)


def _mk_mlp():
    def full(shape):
        return pl.BlockSpec(shape, lambda p, i: tuple(0 for _ in shape))

    return pl.pallas_call(
        _mlp_body,
        grid=(3, NT),
        in_specs=[
            pl.BlockSpec((TB, GD), lambda p, i: (jnp.where(p == 0, i, 0), 0)),
            full((GD, H1)), full((1, H1)), full((1, H1)), full((1, H1)),
            full((1, H1)), full((1, H1)),
            full((H1, H2)), full((1, H2)), full((1, H2)), full((1, H2)),
            full((1, H2)), full((1, H2)),
            full((H2, 1)), full((1, 1)),
        ],
        out_specs=pl.BlockSpec((TB, 1), lambda p, i: (i, 0)),
        out_shape=jax.ShapeDtypeStruct((B, 1), jnp.float32),
        scratch_shapes=[
            pltpu.VMEM((B, H1), jnp.float32),
            pltpu.VMEM((B, H2), jnp.float32),
            pltpu.VMEM((2, H1), jnp.float32),
            pltpu.VMEM((2, H2), jnp.float32),
            pltpu.VMEM((2, H1), jnp.float32),
            pltpu.VMEM((2, H2), jnp.float32),
        ],
    )


_mlp = _mk_mlp()


def kernel(idx, tables, W1, b1, g1a, be1a, g1b, be1b, W2, b2, g2a, be2a,
           g2b, be2b, W3, b3):
    idx2d = idx.reshape(_NIDX // 128, 128)
    tab = tables.reshape(G * V, D)
    x = _mk_gather()(idx2d, tab).reshape(B, GD)
    r = lambda a: a.reshape(1, -1)
    return _mlp(x, W1, r(b1), r(g1a), r(be1a), r(g1b), r(be1b),
                W2, r(b2), r(g2a), r(be2a), r(g2b), r(be2b), W3, r(b3))
